# baseline (speedup 1.0000x reference)
import functools

import jax
import jax.numpy as jnp
from jax.experimental import pallas as pl
from jax.experimental.pallas import tpu as pltpu
from jax.experimental.pallas import tpu_sc as plsc

BLK = 512
FBLK = 512


def _route_body(blk, t_ref, nxt_ref, invp_ref, cnt_ref):
    t = t_ref[...]
    nxt = nxt_ref[...]
    rows, lanes = t.shape
    n = rows * lanes
    r_iota = jax.lax.broadcasted_iota(jnp.int32, t.shape, 0)
    c_iota = jax.lax.broadcasted_iota(jnp.int32, t.shape, 1)
    pos = r_iota * lanes + c_iota
    seq_last = (pos % 4096) == 4095
    vis = (t == 1) & (nxt == 1) & jnp.logical_not(seq_last)
    v = vis.astype(jnp.float32)
    ii = jax.lax.broadcasted_iota(jnp.int32, (lanes, lanes), 0)
    jj = jax.lax.broadcasted_iota(jnp.int32, (lanes, lanes), 1)
    tri = (ii <= jj).astype(jnp.float32)
    intra = jax.lax.dot_general(v, tri, (((1,), (0,)), ((), ())),
                                precision=jax.lax.Precision.HIGHEST)
    row_sum = intra[:, lanes - 1:lanes]
    ri = jax.lax.broadcasted_iota(jnp.int32, (rows, rows), 0)
    rj = jax.lax.broadcasted_iota(jnp.int32, (rows, rows), 1)
    low = (rj < ri).astype(jnp.float32)
    offs = jax.lax.dot_general(low, row_sum, (((1,), (0,)), ((), ())),
                               precision=jax.lax.Precision.HIGHEST)
    excl_v = offs + intra - v
    nv = offs[rows - 1, 0] + row_sum[rows - 1, 0]
    a = jnp.ceil(nv / blk) * blk
    excl_l = pos.astype(jnp.float32) - excl_v
    slot = jnp.where(vis, excl_v, a + excl_l)
    invp_ref[...] = slot.astype(jnp.int32)
    cnt_ref[...] = jnp.full(cnt_ref.shape, nv.astype(jnp.int32))


def _route(t64, n64, blk):
    rows, lanes = t64.shape
    return pl.pallas_call(
        functools.partial(_route_body, blk),
        out_shape=(
            jax.ShapeDtypeStruct((rows, lanes), jnp.int32),
            jax.ShapeDtypeStruct((8, 128), jnp.int32),
        ),
    )(t64, n64)


IWIN = 128
CW = 256


def _sc_scatter(x, idx, cap):
    n, d = x.shape
    mesh = plsc.VectorSubcoreMesh(core_axis_name="core",
                                  subcore_axis_name="subcore")

    @pl.kernel(out_type=jax.ShapeDtypeStruct((cap, d), x.dtype), mesh=mesh)
    def k(x_hbm, i_hbm, o_hbm):
        def body(ids, x_vmem, i_vmem):
            _, j = ids
            pltpu.sync_copy(x_vmem,
                            o_hbm.at[i_vmem.at[0], pl.ds(j * CW, CW)])

        pltpu.emit_pipeline(
            body,
            grid=(n // IWIN, d // CW),
            in_specs=[
                pl.BlockSpec((IWIN, CW), lambda i, j: (i, j)),
                pl.BlockSpec((1, IWIN), lambda i, j: (0, i)),
            ],
            out_specs=[],
            core_axis_name=("core", "subcore"),
            dimension_semantics=(pltpu.PARALLEL, pltpu.PARALLEL),
            _explicit_indices=True,
        )(x_hbm, i_hbm)

    return k(x, idx)


def _sc_gather(y, idx, n):
    cap, d = y.shape
    mesh = plsc.VectorSubcoreMesh(core_axis_name="core",
                                  subcore_axis_name="subcore")

    @pl.kernel(out_type=jax.ShapeDtypeStruct((n, d), y.dtype), mesh=mesh)
    def k(y_hbm, i_hbm, o_hbm):
        def body(ids, i_vmem, o_vmem):
            _, j = ids
            pltpu.sync_copy(y_hbm.at[i_vmem.at[0], pl.ds(j * CW, CW)],
                            o_vmem)

        pltpu.emit_pipeline(
            body,
            grid=(n // IWIN, d // CW),
            in_specs=[pl.BlockSpec((1, IWIN), lambda i, j: (0, i))],
            out_specs=[pl.BlockSpec((IWIN, CW), lambda i, j: (i, j))],
            core_axis_name=("core", "subcore"),
            dimension_semantics=(pltpu.PARALLEL, pltpu.PARALLEL),
            _explicit_indices=True,
        )(i_hbm, o_hbm)

    return k(y, idx)


CH = 256


def _mlp_body(f, expert_ref, x_ref, g_ref, u_ref, d_ref, o_ref):
    j = pl.program_id(1)
    xb = x_ref[...]
    parts = []
    for c in range(FBLK // CH):
        base = j * FBLK + c * CH
        gt = g_ref[0, :, c * CH:(c + 1) * CH]
        ut = u_ref[0, :, c * CH:(c + 1) * CH]
        dt = d_ref[0, c * CH:(c + 1) * CH, :]
        g = jnp.dot(xb, gt, preferred_element_type=jnp.float32)
        u = jnp.dot(xb, ut, preferred_element_type=jnp.float32)
        h = jax.nn.silu(g) * u
        col = jax.lax.broadcasted_iota(jnp.int32, h.shape, 1) + base
        h = jnp.where(col < f, h, 0.0).astype(jnp.bfloat16)
        drow = jax.lax.broadcasted_iota(jnp.int32, dt.shape, 0) + base
        dm = jnp.where(drow < f, dt, jnp.bfloat16(0.0))
        parts.append(jnp.dot(h, dm, preferred_element_type=jnp.float32))
    part = sum(parts)

    @pl.when(j == 0)
    def _():
        o_ref[...] = part

    @pl.when(j > 0)
    def _():
        o_ref[...] += part


def _grouped_mlp(block_expert, xs, w_gate, w_up, w_down):
    cap, d = xs.shape
    f = w_gate.shape[2]
    nsb = cap // BLK
    nf = (f + FBLK - 1) // FBLK

    def xmap(sb, j, e):
        return (sb, 0)

    def gu(sb, j, e):
        return (e[sb], 0, j)

    def dn(sb, j, e):
        return (e[sb], j, 0)

    grid_spec = pltpu.PrefetchScalarGridSpec(
        num_scalar_prefetch=1,
        grid=(nsb, nf),
        in_specs=[
            pl.BlockSpec((BLK, d), xmap),
            pl.BlockSpec((1, d, FBLK), gu),
            pl.BlockSpec((1, d, FBLK), gu),
            pl.BlockSpec((1, FBLK, d), dn),
        ],
        out_specs=pl.BlockSpec((BLK, d), xmap),
    )
    return pl.pallas_call(
        functools.partial(_mlp_body, f),
        grid_spec=grid_spec,
        out_shape=jax.ShapeDtypeStruct((cap, d), jnp.float32),
        compiler_params=pltpu.CompilerParams(
            dimension_semantics=("arbitrary", "arbitrary")),
    )(block_expert, xs, w_gate, w_up, w_down)


def kernel(hidden_states, token_type_ids, v_gate, v_up, v_down,
           l_gate, l_up, l_down):
    b, l, d = hidden_states.shape
    n = b * l
    cap = n + BLK
    nsb = cap // BLK

    tt = token_type_ids.astype(jnp.int32).reshape(-1)
    nxt = jnp.concatenate([tt[1:], jnp.zeros((1,), jnp.int32)])
    t64 = tt.reshape(n // 128, 128)
    n64 = nxt.reshape(n // 128, 128)
    invp, cnt = _route(t64, n64, BLK)

    nv = cnt[0, 0]
    a_blocks = (nv + BLK - 1) // BLK
    block_expert = (jnp.arange(nsb, dtype=jnp.int32) >= a_blocks).astype(
        jnp.int32)

    idx = invp.reshape(1, n)
    xs = _sc_scatter(hidden_states.reshape(n, d), idx, cap)
    w_gate = jnp.stack([v_gate, l_gate]).astype(jnp.bfloat16)
    w_up = jnp.stack([v_up, l_up]).astype(jnp.bfloat16)
    w_down = jnp.stack([v_down, l_down]).astype(jnp.bfloat16)
    ys = _grouped_mlp(block_expert, xs.astype(jnp.bfloat16),
                      w_gate, w_up, w_down)
    out = _sc_gather(ys, idx, n)
    return out.reshape(b, l, d)


# speedup vs baseline: 1.0496x; 1.0496x over previous
import functools

import jax
import jax.numpy as jnp
from jax.experimental import pallas as pl
from jax.experimental.pallas import tpu as pltpu
from jax.experimental.pallas import tpu_sc as plsc

BLK = 1024
FBLK = 256


def _route_body(blk, t_ref, nxt_ref, invp_ref, cnt_ref):
    t = t_ref[...]
    nxt = nxt_ref[...]
    rows, lanes = t.shape
    n = rows * lanes
    r_iota = jax.lax.broadcasted_iota(jnp.int32, t.shape, 0)
    c_iota = jax.lax.broadcasted_iota(jnp.int32, t.shape, 1)
    pos = r_iota * lanes + c_iota
    seq_last = (pos % 4096) == 4095
    vis = (t == 1) & (nxt == 1) & jnp.logical_not(seq_last)
    v = vis.astype(jnp.float32)
    ii = jax.lax.broadcasted_iota(jnp.int32, (lanes, lanes), 0)
    jj = jax.lax.broadcasted_iota(jnp.int32, (lanes, lanes), 1)
    tri = (ii <= jj).astype(jnp.float32)
    intra = jax.lax.dot_general(v, tri, (((1,), (0,)), ((), ())),
                                precision=jax.lax.Precision.HIGHEST)
    row_sum = intra[:, lanes - 1:lanes]
    ri = jax.lax.broadcasted_iota(jnp.int32, (rows, rows), 0)
    rj = jax.lax.broadcasted_iota(jnp.int32, (rows, rows), 1)
    low = (rj < ri).astype(jnp.float32)
    offs = jax.lax.dot_general(low, row_sum, (((1,), (0,)), ((), ())),
                               precision=jax.lax.Precision.HIGHEST)
    excl_v = offs + intra - v
    nv = offs[rows - 1, 0] + row_sum[rows - 1, 0]
    a = jnp.ceil(nv / blk) * blk
    excl_l = pos.astype(jnp.float32) - excl_v
    slot = jnp.where(vis, excl_v, a + excl_l)
    invp_ref[...] = slot.astype(jnp.int32)
    cnt_ref[...] = jnp.full(cnt_ref.shape, nv.astype(jnp.int32))


def _route(t64, n64, blk):
    rows, lanes = t64.shape
    return pl.pallas_call(
        functools.partial(_route_body, blk),
        out_shape=(
            jax.ShapeDtypeStruct((rows, lanes), jnp.int32),
            jax.ShapeDtypeStruct((8, 128), jnp.int32),
        ),
    )(t64, n64)


IWIN = 128
CW = 256


def _sc_scatter(x, idx, cap):
    n, d = x.shape
    mesh = plsc.VectorSubcoreMesh(core_axis_name="core",
                                  subcore_axis_name="subcore")

    @pl.kernel(out_type=jax.ShapeDtypeStruct((cap, d), x.dtype), mesh=mesh)
    def k(x_hbm, i_hbm, o_hbm):
        def body(ids, x_vmem, i_vmem):
            _, j = ids
            pltpu.sync_copy(x_vmem,
                            o_hbm.at[i_vmem.at[0], pl.ds(j * CW, CW)])

        pltpu.emit_pipeline(
            body,
            grid=(n // IWIN, d // CW),
            in_specs=[
                pl.BlockSpec((IWIN, CW), lambda i, j: (i, j)),
                pl.BlockSpec((1, IWIN), lambda i, j: (0, i)),
            ],
            out_specs=[],
            core_axis_name=("core", "subcore"),
            dimension_semantics=(pltpu.PARALLEL, pltpu.PARALLEL),
            _explicit_indices=True,
        )(x_hbm, i_hbm)

    return k(x, idx)


def _sc_gather(y, idx, n):
    cap, d = y.shape
    mesh = plsc.VectorSubcoreMesh(core_axis_name="core",
                                  subcore_axis_name="subcore")

    @pl.kernel(out_type=jax.ShapeDtypeStruct((n, d), y.dtype), mesh=mesh)
    def k(y_hbm, i_hbm, o_hbm):
        def body(ids, i_vmem, o_vmem):
            _, j = ids
            pltpu.sync_copy(y_hbm.at[i_vmem.at[0], pl.ds(j * CW, CW)],
                            o_vmem)

        pltpu.emit_pipeline(
            body,
            grid=(n // IWIN, d // CW),
            in_specs=[pl.BlockSpec((1, IWIN), lambda i, j: (0, i))],
            out_specs=[pl.BlockSpec((IWIN, CW), lambda i, j: (i, j))],
            core_axis_name=("core", "subcore"),
            dimension_semantics=(pltpu.PARALLEL, pltpu.PARALLEL),
            _explicit_indices=True,
        )(i_hbm, o_hbm)

    return k(y, idx)


def _mlp_body(f, expert_ref, x_ref, vg_ref, lg_ref, vu_ref, lu_ref, vd_ref,
              ld_ref, o_ref):
    sb = pl.program_id(0)
    j = pl.program_id(1)
    e = expert_ref[sb]
    xb = x_ref[...]
    gate = jnp.where(e == 0, vg_ref[...], lg_ref[...]).astype(jnp.bfloat16)
    up = jnp.where(e == 0, vu_ref[...], lu_ref[...]).astype(jnp.bfloat16)
    down = jnp.where(e == 0, vd_ref[...], ld_ref[...]).astype(jnp.bfloat16)
    g = jnp.dot(xb, gate, preferred_element_type=jnp.float32)
    u = jnp.dot(xb, up, preferred_element_type=jnp.float32)
    h = jax.nn.silu(g) * u
    col = jax.lax.broadcasted_iota(jnp.int32, h.shape, 1) + j * FBLK
    h = jnp.where(col < f, h, 0.0).astype(jnp.bfloat16)
    drow = jax.lax.broadcasted_iota(jnp.int32, down.shape, 0) + j * FBLK
    down = jnp.where(drow < f, down, jnp.bfloat16(0.0))
    part = jnp.dot(h, down, preferred_element_type=jnp.float32)

    @pl.when(j == 0)
    def _():
        o_ref[...] = part

    @pl.when(j > 0)
    def _():
        o_ref[...] += part


def _grouped_mlp(block_expert, xs, v_gate, v_up, v_down, l_gate, l_up,
                 l_down):
    cap, d = xs.shape
    f = v_gate.shape[1]
    nsb = cap // BLK
    nf = (f + FBLK - 1) // FBLK

    def xmap(sb, j, e):
        return (sb, 0)

    def gu(which):
        def m(sb, j, e):
            return (0, jnp.where(e[sb] == which, j, 0))
        return m

    def dn(which):
        def m(sb, j, e):
            return (jnp.where(e[sb] == which, j, 0), 0)
        return m

    grid_spec = pltpu.PrefetchScalarGridSpec(
        num_scalar_prefetch=1,
        grid=(nsb, nf),
        in_specs=[
            pl.BlockSpec((BLK, d), xmap),
            pl.BlockSpec((d, FBLK), gu(0)),
            pl.BlockSpec((d, FBLK), gu(1)),
            pl.BlockSpec((d, FBLK), gu(0)),
            pl.BlockSpec((d, FBLK), gu(1)),
            pl.BlockSpec((FBLK, d), dn(0)),
            pl.BlockSpec((FBLK, d), dn(1)),
        ],
        out_specs=pl.BlockSpec((BLK, d), xmap),
    )
    return pl.pallas_call(
        functools.partial(_mlp_body, f),
        grid_spec=grid_spec,
        out_shape=jax.ShapeDtypeStruct((cap, d), jnp.float32),
        compiler_params=pltpu.CompilerParams(
            dimension_semantics=("arbitrary", "arbitrary")),
    )(block_expert, xs, v_gate, l_gate, v_up, l_up, v_down, l_down)


def kernel(hidden_states, token_type_ids, v_gate, v_up, v_down,
           l_gate, l_up, l_down):
    b, l, d = hidden_states.shape
    n = b * l
    cap = n + BLK
    nsb = cap // BLK

    tt = token_type_ids.astype(jnp.int32).reshape(-1)
    nxt = jnp.concatenate([tt[1:], jnp.zeros((1,), jnp.int32)])
    t64 = tt.reshape(n // 128, 128)
    n64 = nxt.reshape(n // 128, 128)
    invp, cnt = _route(t64, n64, BLK)

    nv = cnt[0, 0]
    a_blocks = (nv + BLK - 1) // BLK
    block_expert = (jnp.arange(nsb, dtype=jnp.int32) >= a_blocks).astype(
        jnp.int32)

    idx = invp.reshape(1, n)
    xs = _sc_scatter(hidden_states.reshape(n, d), idx, cap)
    ys = _grouped_mlp(block_expert, xs.astype(jnp.bfloat16),
                      v_gate, v_up, v_down, l_gate, l_up, l_down)
    out = _sc_gather(ys, idx, n)
    return out.reshape(b, l, d)
